# Initial kernel scaffold; baseline (speedup 1.0000x reference)
import jax
import jax.numpy as jnp
from jax.experimental import pallas as pl


def kernel(x, hyperedge_index, weight, att):
    raise NotImplementedError("write your pallas kernel here")


# speedup vs baseline: 10.5263x; 10.5263x over previous
import functools

import jax
import jax.numpy as jnp
from jax import lax
from jax.experimental import pallas as pl
from jax.experimental.pallas import tpu as pltpu
from jax.experimental.pallas import tpu_sc as plsc

N = 10000
M = 512
E = 160000
D = 128
GAMMA = 4.2
NEG_SLOPE = 0.2

BN = 1000
GRID = N // BN

NCHUNK = 4
CROWS = N // NCHUNK
CELEMS = CROWS * M
NTILES = 16
STRIPE = CELEMS // NTILES
EPT = E // NTILES
NBATCH = 79
PADEPT = NBATCH * 128 + 16

_HIGH = jax.lax.Precision.HIGHEST


def _tc_xw_body(x_ref, w_ref, a1_ref, xw_ref, ns_ref):
    xw = lax.dot_general(x_ref[...].astype(jnp.bfloat16),
                         w_ref[...].astype(jnp.bfloat16),
                         (((1,), (0,)), ((), ())),
                         preferred_element_type=jnp.float32)
    xw_ref[...] = xw
    ns = jnp.sum(xw * a1_ref[...], axis=1)
    ns_ref[...] = ns.reshape(1, 1, BN)


def _tc_xw(x2d, weight, att1):
    return pl.pallas_call(
        _tc_xw_body,
        grid=(GRID,),
        in_specs=[
            pl.BlockSpec((BN, D), lambda i: (i, 0)),
            pl.BlockSpec((D, D), lambda i: (0, 0)),
            pl.BlockSpec((1, D), lambda i: (0, 0)),
        ],
        out_specs=[
            pl.BlockSpec((BN, D), lambda i: (i, 0)),
            pl.BlockSpec((1, 1, BN), lambda i: (i, 0, 0)),
        ],
        out_shape=[
            jax.ShapeDtypeStruct((N, D), jnp.float32),
            jax.ShapeDtypeStruct((GRID, 1, BN), jnp.float32),
        ],
    )(x2d, weight, att1)


def _tc_esum_body(ct_ref, xw_ref, esum_ref, de_ref, dn_ref, s1_ref):
    i = pl.program_id(0)
    ct = ct_ref[...]
    xw = xw_ref[...]

    @pl.when(i == 0)
    def _():
        esum_ref[...] = jnp.zeros_like(esum_ref)
        de_ref[...] = jnp.zeros_like(de_ref)
        s1_ref[...] = jnp.zeros_like(s1_ref)

    part = lax.dot_general(ct, xw, (((0,), (0,)), ((), ())), precision=_HIGH)
    esum_ref[...] += part
    colsum = jnp.sum(ct, axis=0)
    de_ref[...] += jnp.broadcast_to(colsum.reshape(1, M), (8, M))
    dn = jnp.sum(ct, axis=1)
    dn_ref[...] = dn.reshape(1, 1, BN)
    s1_ref[...] += jnp.sum(dn * jnp.sum(xw, axis=1)).reshape(1, 1)


def _tc_esum(ct, xw):
    return pl.pallas_call(
        _tc_esum_body,
        grid=(GRID,),
        in_specs=[
            pl.BlockSpec((BN, M), lambda i: (i, 0)),
            pl.BlockSpec((BN, D), lambda i: (i, 0)),
        ],
        out_specs=[
            pl.BlockSpec((M, D), lambda i: (0, 0)),
            pl.BlockSpec((8, M), lambda i: (0, 0)),
            pl.BlockSpec((1, 1, BN), lambda i: (i, 0, 0)),
            pl.BlockSpec((1, 1), lambda i: (0, 0)),
        ],
        out_shape=[
            jax.ShapeDtypeStruct((M, D), jnp.float32),
            jax.ShapeDtypeStruct((8, M), jnp.float32),
            jax.ShapeDtypeStruct((GRID, 1, BN), jnp.float32),
            jax.ShapeDtypeStruct((1, 1), jnp.float32),
        ],
    )(ct, xw)


def _tc_small_body(ct_ref, ns_ref, esum_ref, de_ref, a2_ref,
                   es_ref, c_ref, bn_ref, lh_ref, s2_ref):
    i = pl.program_id(0)

    @pl.when(i == 0)
    def _():
        esum = esum_ref[...]
        es = jnp.sum(esum * a2_ref[...], axis=1)
        es_ref[...] = es.reshape(1, M)
        de = de_ref[0:1, :]
        bn_ref[...] = jnp.where(de > 0, 1.0 / jnp.where(de > 0, de, 1.0), 0.0)
        s2_ref[...] = jnp.sum(de * jnp.sum(esum, axis=1).reshape(1, M)).reshape(1, 1)
        n2 = jnp.sum(esum * esum, axis=1)
        ip = lax.dot_general(esum.astype(jnp.bfloat16),
                             esum.astype(jnp.bfloat16),
                             (((1,), (1,)), ((), ())),
                             preferred_element_type=jnp.float32)
        nrm = jnp.sqrt(n2 + 1e-12)
        cos = ip / (nrm[:, None] * nrm[None, :])
        d2 = jnp.maximum(n2[:, None] + n2[None, :] - 2.0 * ip, 0.0)
        dist = jnp.sqrt(d2 + 1e-12)
        li = cos * dist + (1.0 - cos) * jnp.maximum(GAMMA - dist, 0.0)
        lh_ref[...] = (jnp.sum(jnp.abs(li)) / float((M + 1) ** 2)).reshape(1, 1)

    ct = ct_ref[...]
    es_row = es_ref[...]
    raw = jnp.max(jnp.where(ct > 0, es_row, -1e30), axis=1)
    nsb = ns_ref[...].reshape(BN)
    t = nsb + raw
    c = jnp.where(raw > -1e29, jnp.where(t > 0, t, NEG_SLOPE * t), 0.0)
    c_ref[...] = c.reshape(1, 1, BN)


def _tc_small(ct, ns3, esum, de8, att2):
    return pl.pallas_call(
        _tc_small_body,
        grid=(GRID,),
        in_specs=[
            pl.BlockSpec((BN, M), lambda i: (i, 0)),
            pl.BlockSpec((1, 1, BN), lambda i: (i, 0, 0)),
            pl.BlockSpec((M, D), lambda i: (0, 0)),
            pl.BlockSpec((8, M), lambda i: (0, 0)),
            pl.BlockSpec((1, D), lambda i: (0, 0)),
        ],
        out_specs=[
            pl.BlockSpec((1, M), lambda i: (0, 0)),
            pl.BlockSpec((1, 1, BN), lambda i: (i, 0, 0)),
            pl.BlockSpec((1, M), lambda i: (0, 0)),
            pl.BlockSpec((1, 1), lambda i: (0, 0)),
            pl.BlockSpec((1, 1), lambda i: (0, 0)),
        ],
        out_shape=[
            jax.ShapeDtypeStruct((1, M), jnp.float32),
            jax.ShapeDtypeStruct((GRID, 1, BN), jnp.float32),
            jax.ShapeDtypeStruct((1, M), jnp.float32),
            jax.ShapeDtypeStruct((1, 1), jnp.float32),
            jax.ShapeDtypeStruct((1, 1), jnp.float32),
        ],
    )(ct, ns3, esum, de8, att2)


def _tc_oute_body(et_ref, xw_ref, bn_ref, oute_ref):
    i = pl.program_id(0)
    et = et_ref[...]
    asum = jnp.sum(et, axis=1) + 1e-16
    at = et / asum[:, None]
    part = lax.dot_general(at, xw_ref[...], (((0,), (0,)), ((), ())),
                           precision=_HIGH)

    @pl.when(i == 0)
    def _():
        oute_ref[...] = jnp.zeros_like(oute_ref)

    oute_ref[...] += part

    @pl.when(i == pl.num_programs(0) - 1)
    def _():
        oute_ref[...] = oute_ref[...] * bn_ref[...]


def _tc_oute(et, xw, bn_col):
    return pl.pallas_call(
        _tc_oute_body,
        grid=(GRID,),
        in_specs=[
            pl.BlockSpec((BN, M), lambda i: (i, 0)),
            pl.BlockSpec((BN, D), lambda i: (i, 0)),
            pl.BlockSpec((M, 1), lambda i: (0, 0)),
        ],
        out_specs=pl.BlockSpec((M, D), lambda i: (0, 0)),
        out_shape=jax.ShapeDtypeStruct((M, D), jnp.float32),
    )(et, xw, bn_col)


def _tc_outn_body(et_ref, dn_ref, oute_ref, outn_ref):
    et = et_ref[...]
    asum = jnp.sum(et, axis=1) + 1e-16
    at = et / asum[:, None]
    r = lax.dot_general(at, oute_ref[...], (((1,), (0,)), ((), ())),
                        precision=_HIGH)
    outn_ref[...] = r * dn_ref[...]


def _tc_outn(et, dn_col, oute):
    return pl.pallas_call(
        _tc_outn_body,
        grid=(GRID,),
        in_specs=[
            pl.BlockSpec((BN, M), lambda i: (i, 0)),
            pl.BlockSpec((BN, 1), lambda i: (i, 0)),
            pl.BlockSpec((M, D), lambda i: (0, 0)),
        ],
        out_specs=pl.BlockSpec((BN, D), lambda i: (i, 0)),
        out_shape=jax.ShapeDtypeStruct((N, D), jnp.float32),
    )(et, dn_col, oute)


def _sc_zero_stripe(zbuf, chunk, sid):
    for i in range(125):
        zbuf[pl.ds(i * 16, 16)] = jnp.zeros((16,), jnp.float32)
    for i in range(STRIPE // 2000):
        pltpu.sync_copy(zbuf, chunk.at[pl.ds(sid * STRIPE + i * 2000, 2000)])


def _sc_scan_round(nbuf, ebuf, idxrow, valrow, chunk, node_lo, value_fn):
    lane = lax.iota(jnp.int32, 16)

    def batch(j, carry):
        for kk in range(8):
            off = j * 128 + kk * 16
            nv = nbuf[pl.ds(off, 16)]
            ev = ebuf[pl.ds(off, 16)]
            valid = (off + lane) < EPT
            rel = nv - node_lo
            inr = valid & (rel >= 0) & (rel < CROWS)
            flat = rel * M + ev
            idxrow[0, pl.ds(kk * 16, 16)] = jnp.where(inr, flat, 0)
            vals = value_fn(nv, ev)
            valrow[0, pl.ds(kk * 16, 16)] = jnp.where(
                inr, vals, jnp.zeros((16,), jnp.float32))
        pltpu.sync_copy(valrow.at[0], chunk.at[idxrow.at[0]], add=True)
        return carry

    lax.fori_loop(0, NBATCH, batch, 0)


def _sc_accumulate(node_hbm, edge_hbm, out_hbm, nbuf, ebuf, idxrow, valrow,
                   zbuf, chunk, value_fn):
    cid = lax.axis_index("c")
    sid = lax.axis_index("s")
    pltpu.sync_copy(node_hbm.at[pl.ds(sid * EPT, EPT)], nbuf.at[pl.ds(0, EPT)])
    pltpu.sync_copy(edge_hbm.at[pl.ds(sid * EPT, EPT)], ebuf.at[pl.ds(0, EPT)])
    for i in range((PADEPT - EPT) // 16):
        nbuf[pl.ds(EPT + i * 16, 16)] = jnp.zeros((16,), jnp.int32)
        ebuf[pl.ds(EPT + i * 16, 16)] = jnp.zeros((16,), jnp.int32)
    for rnd in range(NCHUNK // 2):
        chunk_id = rnd * 2 + cid
        node_lo = chunk_id * CROWS
        _sc_zero_stripe(zbuf, chunk, sid)
        plsc.subcore_barrier()
        _sc_scan_round(nbuf, ebuf, idxrow, valrow, chunk, node_lo, value_fn)
        plsc.subcore_barrier()
        pltpu.sync_copy(
            chunk.at[pl.ds(sid * STRIPE, STRIPE)],
            out_hbm.at[pl.ds(chunk_id * CELEMS + sid * STRIPE, STRIPE)])
        plsc.subcore_barrier()


def _sc_mesh():
    return plsc.VectorSubcoreMesh(core_axis_name="c", subcore_axis_name="s")


def _sc_count_body(node_hbm, edge_hbm, out_hbm, nbuf, ebuf, idxrow, valrow,
                   zbuf, chunk):
    def ones(nv, ev):
        return jnp.ones((16,), jnp.float32)

    _sc_accumulate(node_hbm, edge_hbm, out_hbm, nbuf, ebuf, idxrow, valrow,
                   zbuf, chunk, ones)


def _sc_count(node, edge):
    k = pl.kernel(
        _sc_count_body,
        mesh=_sc_mesh(),
        out_type=jax.ShapeDtypeStruct((N * M,), jnp.float32),
        compiler_params=pltpu.CompilerParams(needs_layout_passes=False),
        scratch_types=[
            pltpu.VMEM((PADEPT,), jnp.int32),
            pltpu.VMEM((PADEPT,), jnp.int32),
            pltpu.VMEM((1, 128), jnp.int32),
            pltpu.VMEM((1, 128), jnp.float32),
            pltpu.VMEM((2000,), jnp.float32),
            pltpu.VMEM_SHARED((CELEMS,), jnp.float32),
        ],
    )
    return k(node, edge)


def _sc_expw_body(node_hbm, edge_hbm, ns_hbm, c_hbm, es_hbm, out_hbm,
                  nbuf, ebuf, idxrow, valrow, zbuf, chunk, ns_t, c_t, es_t):
    pltpu.sync_copy(ns_hbm, ns_t)
    pltpu.sync_copy(c_hbm, c_t)
    pltpu.sync_copy(es_hbm, es_t)

    def expw(nv, ev):
        nsv = plsc.load_gather(ns_t, [nv])
        cv = plsc.load_gather(c_t, [nv])
        esv = plsc.load_gather(es_t, [ev])
        s = nsv + esv
        a = jnp.where(s > 0, s, NEG_SLOPE * s)
        return jnp.exp(a - cv)

    _sc_accumulate(node_hbm, edge_hbm, out_hbm, nbuf, ebuf, idxrow, valrow,
                   zbuf, chunk, expw)


def _sc_expw(node, edge, ns, c, es):
    k = pl.kernel(
        _sc_expw_body,
        mesh=_sc_mesh(),
        out_type=jax.ShapeDtypeStruct((N * M,), jnp.float32),
        compiler_params=pltpu.CompilerParams(needs_layout_passes=False),
        scratch_types=[
            pltpu.VMEM((PADEPT,), jnp.int32),
            pltpu.VMEM((PADEPT,), jnp.int32),
            pltpu.VMEM((1, 128), jnp.int32),
            pltpu.VMEM((1, 128), jnp.float32),
            pltpu.VMEM((2000,), jnp.float32),
            pltpu.VMEM_SHARED((CELEMS,), jnp.float32),
            pltpu.VMEM((N,), jnp.float32),
            pltpu.VMEM((N,), jnp.float32),
            pltpu.VMEM((M,), jnp.float32),
        ],
    )
    return k(node, edge, ns, c, es)


def kernel(x, hyperedge_index, weight, att):
    node = hyperedge_index[0].astype(jnp.int32)
    edge = hyperedge_index[1].astype(jnp.int32)
    x2d = x[0]
    att1 = att[0, :, :D]
    att2 = att[0, :, D:]

    xw, ns3 = _tc_xw(x2d, weight, att1)
    ct = _sc_count(node, edge).reshape(N, M)
    esum, de8, dn3, s1 = _tc_esum(ct, xw)
    es2, c3, bn2, lh, s2 = _tc_small(ct, ns3, esum, de8, att2)
    et = _sc_expw(node, edge, ns3.reshape(N), c3.reshape(N),
                  es2.reshape(M)).reshape(N, M)
    oute = _tc_oute(et, xw, bn2.reshape(M, 1))
    outn = _tc_outn(et, dn3.reshape(N, 1), oute)

    constrain = jnp.abs((s1[0, 0] - s2[0, 0]) / float(E * D)) + lh[0, 0]
    return outn[None], constrain


# speedup vs baseline: 23.5351x; 2.2358x over previous
import functools

import jax
import jax.numpy as jnp
from jax import lax
from jax.experimental import pallas as pl
from jax.experimental.pallas import tpu as pltpu
from jax.experimental.pallas import tpu_sc as plsc

N = 10000
M = 512
E = 160000
D = 128
GAMMA = 4.2
NEG_SLOPE = 0.2

BN = 1000
GRID = N // BN

NCHUNK = 8
CROWS = N // NCHUNK
CELEMS = CROWS * M
NTILES = 16
STRIPE = CELEMS // NTILES
NOUT = 8
OSTRIPE = CELEMS // NOUT
EPT = E // NTILES
NBATCH = 79
PADEPT = NBATCH * 128 + 16
NBG = 80

_HIGH = jax.lax.Precision.HIGHEST


def _tc_xw_body(x_ref, w_ref, a1_ref, xw_ref, ns_ref):
    xw = lax.dot_general(x_ref[...].astype(jnp.bfloat16),
                         w_ref[...].astype(jnp.bfloat16),
                         (((1,), (0,)), ((), ())),
                         preferred_element_type=jnp.float32)
    xw_ref[...] = xw
    ns = jnp.sum(xw * a1_ref[...], axis=1)
    ns_ref[...] = ns.reshape(1, 1, BN)


def _tc_xw(x2d, weight, att1):
    return pl.pallas_call(
        _tc_xw_body,
        grid=(GRID,),
        in_specs=[
            pl.BlockSpec((BN, D), lambda i: (i, 0)),
            pl.BlockSpec((D, D), lambda i: (0, 0)),
            pl.BlockSpec((1, D), lambda i: (0, 0)),
        ],
        out_specs=[
            pl.BlockSpec((BN, D), lambda i: (i, 0)),
            pl.BlockSpec((1, 1, BN), lambda i: (i, 0, 0)),
        ],
        out_shape=[
            jax.ShapeDtypeStruct((N, D), jnp.float32),
            jax.ShapeDtypeStruct((GRID, 1, BN), jnp.float32),
        ],
    )(x2d, weight, att1)


def _tc_esum_body(ct_ref, xw_ref, esum_ref, de_ref, dn_ref, s1_ref):
    i = pl.program_id(0)
    ct = ct_ref[...]
    xw = xw_ref[...]

    @pl.when(i == 0)
    def _():
        esum_ref[...] = jnp.zeros_like(esum_ref)
        de_ref[...] = jnp.zeros_like(de_ref)
        s1_ref[...] = jnp.zeros_like(s1_ref)

    part = lax.dot_general(ct, xw, (((0,), (0,)), ((), ())), precision=_HIGH)
    esum_ref[...] += part
    colsum = jnp.sum(ct, axis=0)
    de_ref[...] += jnp.broadcast_to(colsum.reshape(1, M), (8, M))
    dn = jnp.sum(ct, axis=1)
    dn_ref[...] = dn.reshape(1, 1, BN)
    s1_ref[...] += jnp.sum(dn * jnp.sum(xw, axis=1)).reshape(1, 1)


def _tc_esum(ct, xw):
    return pl.pallas_call(
        _tc_esum_body,
        grid=(GRID,),
        in_specs=[
            pl.BlockSpec((BN, M), lambda i: (i, 0)),
            pl.BlockSpec((BN, D), lambda i: (i, 0)),
        ],
        out_specs=[
            pl.BlockSpec((M, D), lambda i: (0, 0)),
            pl.BlockSpec((8, M), lambda i: (0, 0)),
            pl.BlockSpec((1, 1, BN), lambda i: (i, 0, 0)),
            pl.BlockSpec((1, 1), lambda i: (0, 0)),
        ],
        out_shape=[
            jax.ShapeDtypeStruct((M, D), jnp.float32),
            jax.ShapeDtypeStruct((8, M), jnp.float32),
            jax.ShapeDtypeStruct((GRID, 1, BN), jnp.float32),
            jax.ShapeDtypeStruct((1, 1), jnp.float32),
        ],
    )(ct, xw)


def _tc_small_body(ct_ref, ns_ref, esum_ref, de_ref, a2_ref,
                   es_ref, c_ref, bn_ref, lh_ref, s2_ref):
    i = pl.program_id(0)

    @pl.when(i == 0)
    def _():
        esum = esum_ref[...]
        es = jnp.sum(esum * a2_ref[...], axis=1)
        es_ref[...] = es.reshape(1, M)
        de = de_ref[0:1, :]
        bn_ref[...] = jnp.where(de > 0, 1.0 / jnp.where(de > 0, de, 1.0), 0.0)
        s2_ref[...] = jnp.sum(de * jnp.sum(esum, axis=1).reshape(1, M)).reshape(1, 1)
        n2 = jnp.sum(esum * esum, axis=1)
        ip = lax.dot_general(esum.astype(jnp.bfloat16),
                             esum.astype(jnp.bfloat16),
                             (((1,), (1,)), ((), ())),
                             preferred_element_type=jnp.float32)
        nrm = jnp.sqrt(n2 + 1e-12)
        cos = ip / (nrm[:, None] * nrm[None, :])
        d2 = jnp.maximum(n2[:, None] + n2[None, :] - 2.0 * ip, 0.0)
        dist = jnp.sqrt(d2 + 1e-12)
        li = cos * dist + (1.0 - cos) * jnp.maximum(GAMMA - dist, 0.0)
        lh_ref[...] = (jnp.sum(jnp.abs(li)) / float((M + 1) ** 2)).reshape(1, 1)

    ct = ct_ref[...]
    es_row = es_ref[...]
    raw = jnp.max(jnp.where(ct > 0, es_row, -1e30), axis=1)
    nsb = ns_ref[...].reshape(BN)
    t = nsb + raw
    c = jnp.where(raw > -1e29, jnp.where(t > 0, t, NEG_SLOPE * t), 0.0)
    c_ref[...] = c.reshape(1, 1, BN)


def _tc_small(ct, ns3, esum, de8, att2):
    return pl.pallas_call(
        _tc_small_body,
        grid=(GRID,),
        in_specs=[
            pl.BlockSpec((BN, M), lambda i: (i, 0)),
            pl.BlockSpec((1, 1, BN), lambda i: (i, 0, 0)),
            pl.BlockSpec((M, D), lambda i: (0, 0)),
            pl.BlockSpec((8, M), lambda i: (0, 0)),
            pl.BlockSpec((1, D), lambda i: (0, 0)),
        ],
        out_specs=[
            pl.BlockSpec((1, M), lambda i: (0, 0)),
            pl.BlockSpec((1, 1, BN), lambda i: (i, 0, 0)),
            pl.BlockSpec((1, M), lambda i: (0, 0)),
            pl.BlockSpec((1, 1), lambda i: (0, 0)),
            pl.BlockSpec((1, 1), lambda i: (0, 0)),
        ],
        out_shape=[
            jax.ShapeDtypeStruct((1, M), jnp.float32),
            jax.ShapeDtypeStruct((GRID, 1, BN), jnp.float32),
            jax.ShapeDtypeStruct((1, M), jnp.float32),
            jax.ShapeDtypeStruct((1, 1), jnp.float32),
            jax.ShapeDtypeStruct((1, 1), jnp.float32),
        ],
    )(ct, ns3, esum, de8, att2)


def _tc_oute_body(et_ref, xw_ref, bn_ref, oute_ref):
    i = pl.program_id(0)
    et = et_ref[...]
    asum = jnp.sum(et, axis=1) + 1e-16
    at = et / asum[:, None]
    part = lax.dot_general(at, xw_ref[...], (((0,), (0,)), ((), ())),
                           precision=_HIGH)

    @pl.when(i == 0)
    def _():
        oute_ref[...] = jnp.zeros_like(oute_ref)

    oute_ref[...] += part

    @pl.when(i == pl.num_programs(0) - 1)
    def _():
        oute_ref[...] = oute_ref[...] * bn_ref[...]


def _tc_oute(et, xw, bn_col):
    return pl.pallas_call(
        _tc_oute_body,
        grid=(GRID,),
        in_specs=[
            pl.BlockSpec((BN, M), lambda i: (i, 0)),
            pl.BlockSpec((BN, D), lambda i: (i, 0)),
            pl.BlockSpec((M, 1), lambda i: (0, 0)),
        ],
        out_specs=pl.BlockSpec((M, D), lambda i: (0, 0)),
        out_shape=jax.ShapeDtypeStruct((M, D), jnp.float32),
    )(et, xw, bn_col)


def _tc_outn_body(et_ref, dn_ref, oute_ref, outn_ref):
    et = et_ref[...]
    asum = jnp.sum(et, axis=1) + 1e-16
    at = et / asum[:, None]
    r = lax.dot_general(at, oute_ref[...], (((1,), (0,)), ((), ())),
                        precision=_HIGH)
    outn_ref[...] = r * dn_ref[...]


def _tc_outn(et, dn_col, oute):
    return pl.pallas_call(
        _tc_outn_body,
        grid=(GRID,),
        in_specs=[
            pl.BlockSpec((BN, M), lambda i: (i, 0)),
            pl.BlockSpec((BN, 1), lambda i: (i, 0)),
            pl.BlockSpec((M, D), lambda i: (0, 0)),
        ],
        out_specs=pl.BlockSpec((BN, D), lambda i: (i, 0)),
        out_shape=jax.ShapeDtypeStruct((N, D), jnp.float32),
    )(et, dn_col, oute)


def _sc_zero_stripe(zbuf, chunk, sid):
    for i in range(125):
        zbuf[pl.ds(i * 16, 16)] = jnp.zeros((16,), jnp.float32)
    for i in range(STRIPE // 2000):
        pltpu.sync_copy(zbuf, chunk.at[pl.ds(sid * STRIPE + i * 2000, 2000)])


def _sc_scan_round(nbuf, ebuf, cidx, cvals, chunk, node_lo, value_fn):
    lane = lax.iota(jnp.int32, 16)

    def clear(i, c):
        cidx[pl.ds(i * 16, 16)] = jnp.zeros((16,), jnp.int32)
        cvals[pl.ds(i * 16, 16)] = jnp.zeros((16,), jnp.float32)
        return c

    lax.fori_loop(0, NBG * 8, clear, 0)

    def scan(j, cnt):
        for kk in range(8):
            off = j * 128 + kk * 16
            nv = nbuf[pl.ds(off, 16)]
            ev = ebuf[pl.ds(off, 16)]
            valid = (off + lane) < EPT
            rel = nv - node_lo
            inr = valid & (rel >= 0) & (rel < CROWS)
            flat = rel * M + ev
            vals = value_fn(nv, ev)
            pos = cnt + lax.cumsum(inr.astype(jnp.int32), axis=0) - 1
            plsc.store_scatter(cidx, [pos], flat, mask=inr)
            plsc.store_scatter(cvals, [pos], vals, mask=inr)
            cnt = cnt + jnp.sum(inr.astype(jnp.int32))
        return cnt

    cnt = lax.fori_loop(0, NBATCH, scan, jnp.int32(0))
    nb = (cnt + 127) // 128

    def fire(b, c):
        pltpu.sync_copy(cvals.at[pl.ds(b * 128, 128)],
                        chunk.at[cidx.at[pl.ds(b * 128, 128)]], add=True)
        return c

    lax.fori_loop(0, nb, fire, 0)


def _sc_accumulate(node_hbm, edge_hbm, out_hbm, nbuf, ebuf, cidx, cvals,
                   zbuf, chunk, value_fn):
    cid = lax.axis_index("c")
    sid = lax.axis_index("s")
    pltpu.sync_copy(node_hbm.at[pl.ds(sid * EPT, EPT)], nbuf.at[pl.ds(0, EPT)])
    pltpu.sync_copy(edge_hbm.at[pl.ds(sid * EPT, EPT)], ebuf.at[pl.ds(0, EPT)])
    for i in range((PADEPT - EPT) // 16):
        nbuf[pl.ds(EPT + i * 16, 16)] = jnp.zeros((16,), jnp.int32)
        ebuf[pl.ds(EPT + i * 16, 16)] = jnp.zeros((16,), jnp.int32)
    for rnd in range(NCHUNK // 2):
        chunk_id = rnd * 2 + cid
        node_lo = chunk_id * CROWS
        _sc_zero_stripe(zbuf, chunk, sid)
        plsc.subcore_barrier()
        _sc_scan_round(nbuf, ebuf, cidx, cvals, chunk, node_lo, value_fn)
        plsc.subcore_barrier()

        @pl.when(sid < NOUT)
        def _():
            pltpu.sync_copy(
                chunk.at[pl.ds(sid * OSTRIPE, OSTRIPE)],
                out_hbm.at[pl.ds(chunk_id * CELEMS + sid * OSTRIPE, OSTRIPE)])

        plsc.subcore_barrier()


def _sc_mesh():
    return plsc.VectorSubcoreMesh(core_axis_name="c", subcore_axis_name="s")


def _sc_count_body(node_hbm, edge_hbm, out_hbm, nbuf, ebuf, cidx, cvals,
                   zbuf, chunk):
    def ones(nv, ev):
        return jnp.ones((16,), jnp.float32)

    _sc_accumulate(node_hbm, edge_hbm, out_hbm, nbuf, ebuf, cidx, cvals,
                   zbuf, chunk, ones)


def _sc_count(node, edge):
    k = pl.kernel(
        _sc_count_body,
        mesh=_sc_mesh(),
        out_type=jax.ShapeDtypeStruct((N * M,), jnp.float32),
        compiler_params=pltpu.CompilerParams(needs_layout_passes=False),
        scratch_types=[
            pltpu.VMEM((PADEPT,), jnp.int32),
            pltpu.VMEM((PADEPT,), jnp.int32),
            pltpu.VMEM((NBG * 128,), jnp.int32),
            pltpu.VMEM((NBG * 128,), jnp.float32),
            pltpu.VMEM((2000,), jnp.float32),
            pltpu.VMEM_SHARED((CELEMS,), jnp.float32),
        ],
    )
    return k(node, edge)


def _sc_expw_body(node_hbm, edge_hbm, ns_hbm, c_hbm, es_hbm, out_hbm,
                  nbuf, ebuf, cidx, cvals, zbuf, chunk, ns_t, c_t, es_t):
    pltpu.sync_copy(ns_hbm, ns_t)
    pltpu.sync_copy(c_hbm, c_t)
    pltpu.sync_copy(es_hbm, es_t)

    def expw(nv, ev):
        nsv = plsc.load_gather(ns_t, [nv])
        cv = plsc.load_gather(c_t, [nv])
        esv = plsc.load_gather(es_t, [ev])
        s = nsv + esv
        a = jnp.where(s > 0, s, NEG_SLOPE * s)
        return jnp.exp(a - cv)

    _sc_accumulate(node_hbm, edge_hbm, out_hbm, nbuf, ebuf, cidx, cvals,
                   zbuf, chunk, expw)


def _sc_expw(node, edge, ns, c, es):
    k = pl.kernel(
        _sc_expw_body,
        mesh=_sc_mesh(),
        out_type=jax.ShapeDtypeStruct((N * M,), jnp.float32),
        compiler_params=pltpu.CompilerParams(needs_layout_passes=False),
        scratch_types=[
            pltpu.VMEM((PADEPT,), jnp.int32),
            pltpu.VMEM((PADEPT,), jnp.int32),
            pltpu.VMEM((NBG * 128,), jnp.int32),
            pltpu.VMEM((NBG * 128,), jnp.float32),
            pltpu.VMEM((2000,), jnp.float32),
            pltpu.VMEM_SHARED((CELEMS,), jnp.float32),
            pltpu.VMEM((N,), jnp.float32),
            pltpu.VMEM((N,), jnp.float32),
            pltpu.VMEM((M,), jnp.float32),
        ],
    )
    return k(node, edge, ns, c, es)


def kernel(x, hyperedge_index, weight, att):
    node = hyperedge_index[0].astype(jnp.int32)
    edge = hyperedge_index[1].astype(jnp.int32)
    x2d = x[0]
    att1 = att[0, :, :D]
    att2 = att[0, :, D:]

    xw, ns3 = _tc_xw(x2d, weight, att1)
    ct = _sc_count(node, edge).reshape(N, M)
    esum, de8, dn3, s1 = _tc_esum(ct, xw)
    es2, c3, bn2, lh, s2 = _tc_small(ct, ns3, esum, de8, att2)
    et = _sc_expw(node, edge, ns3.reshape(N), c3.reshape(N),
                  es2.reshape(M)).reshape(N, M)
    oute = _tc_oute(et, xw, bn2.reshape(M, 1))
    outn = _tc_outn(et, dn3.reshape(N, 1), oute)

    constrain = jnp.abs((s1[0, 0] - s2[0, 0]) / float(E * D)) + lh[0, 0]
    return outn[None], constrain


# speedup vs baseline: 25.0659x; 1.0650x over previous
import functools

import jax
import jax.numpy as jnp
from jax import lax
from jax.experimental import pallas as pl
from jax.experimental.pallas import tpu as pltpu
from jax.experimental.pallas import tpu_sc as plsc

N = 10000
M = 512
E = 160000
D = 128
GAMMA = 4.2
NEG_SLOPE = 0.2

BN = 1000
GRID = N // BN

NCHUNK = 8
CROWS = N // NCHUNK
CELEMS = CROWS * M
NTILES = 16
STRIPE = CELEMS // NTILES
NOUT = 8
OSTRIPE = CELEMS // NOUT
EPT = E // NTILES
NBATCH = 79
PADEPT = NBATCH * 128 + 16
NBG = 80

_HIGH = jax.lax.Precision.HIGHEST


def _tc_xw_body(x_ref, w_ref, a1_ref, xw_ref, ns_ref):
    xw = lax.dot_general(x_ref[...].astype(jnp.bfloat16),
                         w_ref[...].astype(jnp.bfloat16),
                         (((1,), (0,)), ((), ())),
                         preferred_element_type=jnp.float32)
    xw_ref[...] = xw
    ns = jnp.sum(xw * a1_ref[...], axis=1)
    ns_ref[...] = ns.reshape(1, 1, BN)


def _tc_xw(x2d, weight, att1):
    return pl.pallas_call(
        _tc_xw_body,
        grid=(GRID,),
        in_specs=[
            pl.BlockSpec((BN, D), lambda i: (i, 0)),
            pl.BlockSpec((D, D), lambda i: (0, 0)),
            pl.BlockSpec((1, D), lambda i: (0, 0)),
        ],
        out_specs=[
            pl.BlockSpec((BN, D), lambda i: (i, 0)),
            pl.BlockSpec((1, 1, BN), lambda i: (i, 0, 0)),
        ],
        out_shape=[
            jax.ShapeDtypeStruct((N, D), jnp.float32),
            jax.ShapeDtypeStruct((GRID, 1, BN), jnp.float32),
        ],
    )(x2d, weight, att1)


def _tc_esum_body(ct_ref, xw_ref, esum_ref, de_ref, dn_ref, s1_ref):
    i = pl.program_id(0)
    ct = ct_ref[...]
    xw = xw_ref[...]

    @pl.when(i == 0)
    def _():
        esum_ref[...] = jnp.zeros_like(esum_ref)
        de_ref[...] = jnp.zeros_like(de_ref)
        s1_ref[...] = jnp.zeros_like(s1_ref)

    part = lax.dot_general(ct, xw, (((0,), (0,)), ((), ())), precision=_HIGH)
    esum_ref[...] += part
    colsum = jnp.sum(ct, axis=0)
    de_ref[...] += jnp.broadcast_to(colsum.reshape(1, M), (8, M))
    dn = jnp.sum(ct, axis=1)
    dn_ref[...] = dn.reshape(1, 1, BN)
    s1_ref[...] += jnp.sum(dn * jnp.sum(xw, axis=1)).reshape(1, 1)


def _tc_esum(ct, xw):
    return pl.pallas_call(
        _tc_esum_body,
        grid=(GRID,),
        in_specs=[
            pl.BlockSpec((BN, M), lambda i: (i, 0)),
            pl.BlockSpec((BN, D), lambda i: (i, 0)),
        ],
        out_specs=[
            pl.BlockSpec((M, D), lambda i: (0, 0)),
            pl.BlockSpec((8, M), lambda i: (0, 0)),
            pl.BlockSpec((1, 1, BN), lambda i: (i, 0, 0)),
            pl.BlockSpec((1, 1), lambda i: (0, 0)),
        ],
        out_shape=[
            jax.ShapeDtypeStruct((M, D), jnp.float32),
            jax.ShapeDtypeStruct((8, M), jnp.float32),
            jax.ShapeDtypeStruct((GRID, 1, BN), jnp.float32),
            jax.ShapeDtypeStruct((1, 1), jnp.float32),
        ],
    )(ct, xw)


def _tc_small_body(ct_ref, ns_ref, esum_ref, de_ref, a2_ref,
                   es_ref, c_ref, bn_ref, lh_ref, s2_ref):
    i = pl.program_id(0)

    @pl.when(i == 0)
    def _():
        esum = esum_ref[...]
        es = jnp.sum(esum * a2_ref[...], axis=1)
        es_ref[...] = es.reshape(1, M)
        de = de_ref[0:1, :]
        bn_ref[...] = jnp.where(de > 0, 1.0 / jnp.where(de > 0, de, 1.0), 0.0)
        s2_ref[...] = jnp.sum(de * jnp.sum(esum, axis=1).reshape(1, M)).reshape(1, 1)
        n2 = jnp.sum(esum * esum, axis=1)
        ip = lax.dot_general(esum.astype(jnp.bfloat16),
                             esum.astype(jnp.bfloat16),
                             (((1,), (1,)), ((), ())),
                             preferred_element_type=jnp.float32)
        nrm = jnp.sqrt(n2 + 1e-12)
        cos = ip / (nrm[:, None] * nrm[None, :])
        d2 = jnp.maximum(n2[:, None] + n2[None, :] - 2.0 * ip, 0.0)
        dist = jnp.sqrt(d2 + 1e-12)
        li = cos * dist + (1.0 - cos) * jnp.maximum(GAMMA - dist, 0.0)
        lh_ref[...] = (jnp.sum(jnp.abs(li)) / float((M + 1) ** 2)).reshape(1, 1)

    ct = ct_ref[...]
    es_row = es_ref[...]
    raw = jnp.max(jnp.where(ct > 0, es_row, -1e30), axis=1)
    nsb = ns_ref[...].reshape(BN)
    t = nsb + raw
    c = jnp.where(raw > -1e29, jnp.where(t > 0, t, NEG_SLOPE * t), 0.0)
    c_ref[...] = c.reshape(1, 1, BN)


def _tc_small(ct, ns3, esum, de8, att2):
    return pl.pallas_call(
        _tc_small_body,
        grid=(GRID,),
        in_specs=[
            pl.BlockSpec((BN, M), lambda i: (i, 0)),
            pl.BlockSpec((1, 1, BN), lambda i: (i, 0, 0)),
            pl.BlockSpec((M, D), lambda i: (0, 0)),
            pl.BlockSpec((8, M), lambda i: (0, 0)),
            pl.BlockSpec((1, D), lambda i: (0, 0)),
        ],
        out_specs=[
            pl.BlockSpec((1, M), lambda i: (0, 0)),
            pl.BlockSpec((1, 1, BN), lambda i: (i, 0, 0)),
            pl.BlockSpec((1, M), lambda i: (0, 0)),
            pl.BlockSpec((1, 1), lambda i: (0, 0)),
            pl.BlockSpec((1, 1), lambda i: (0, 0)),
        ],
        out_shape=[
            jax.ShapeDtypeStruct((1, M), jnp.float32),
            jax.ShapeDtypeStruct((GRID, 1, BN), jnp.float32),
            jax.ShapeDtypeStruct((1, M), jnp.float32),
            jax.ShapeDtypeStruct((1, 1), jnp.float32),
            jax.ShapeDtypeStruct((1, 1), jnp.float32),
        ],
    )(ct, ns3, esum, de8, att2)


def _tc_oute_body(et_ref, xw_ref, bn_ref, oute_ref):
    i = pl.program_id(0)
    et = et_ref[...]
    asum = jnp.sum(et, axis=1) + 1e-16
    at = et / asum[:, None]
    part = lax.dot_general(at, xw_ref[...], (((0,), (0,)), ((), ())),
                           precision=_HIGH)

    @pl.when(i == 0)
    def _():
        oute_ref[...] = jnp.zeros_like(oute_ref)

    oute_ref[...] += part

    @pl.when(i == pl.num_programs(0) - 1)
    def _():
        oute_ref[...] = oute_ref[...] * bn_ref[...]


def _tc_oute(et, xw, bn_col):
    return pl.pallas_call(
        _tc_oute_body,
        grid=(GRID,),
        in_specs=[
            pl.BlockSpec((BN, M), lambda i: (i, 0)),
            pl.BlockSpec((BN, D), lambda i: (i, 0)),
            pl.BlockSpec((M, 1), lambda i: (0, 0)),
        ],
        out_specs=pl.BlockSpec((M, D), lambda i: (0, 0)),
        out_shape=jax.ShapeDtypeStruct((M, D), jnp.float32),
    )(et, xw, bn_col)


def _tc_outn_body(et_ref, dn_ref, oute_ref, outn_ref):
    et = et_ref[...]
    asum = jnp.sum(et, axis=1) + 1e-16
    at = et / asum[:, None]
    r = lax.dot_general(at, oute_ref[...], (((1,), (0,)), ((), ())),
                        precision=_HIGH)
    outn_ref[...] = r * dn_ref[...]


def _tc_outn(et, dn_col, oute):
    return pl.pallas_call(
        _tc_outn_body,
        grid=(GRID,),
        in_specs=[
            pl.BlockSpec((BN, M), lambda i: (i, 0)),
            pl.BlockSpec((BN, 1), lambda i: (i, 0)),
            pl.BlockSpec((M, D), lambda i: (0, 0)),
        ],
        out_specs=pl.BlockSpec((BN, D), lambda i: (i, 0)),
        out_shape=jax.ShapeDtypeStruct((N, D), jnp.float32),
    )(et, dn_col, oute)


def _sc_zero_stripe(zbuf, chunk, sid):
    for i in range(125):
        zbuf[pl.ds(i * 16, 16)] = jnp.zeros((16,), jnp.float32)
    for i in range(STRIPE // 2000):
        pltpu.sync_copy(zbuf, chunk.at[pl.ds(sid * STRIPE + i * 2000, 2000)])


def _sc_scan_round(nbuf, ebuf, cidx, cvals, chunk, sem, node_lo, value_fn):
    lane = lax.iota(jnp.int32, 16)

    def scan(j, cnt):
        for kk in range(8):
            off = j * 128 + kk * 16
            nv = nbuf[pl.ds(off, 16)]
            ev = ebuf[pl.ds(off, 16)]
            valid = (off + lane) < EPT
            rel = nv - node_lo
            inr = valid & (rel >= 0) & (rel < CROWS)
            flat = rel * M + ev
            vals = value_fn(nv, ev)
            pos = cnt + lax.cumsum(inr.astype(jnp.int32), axis=0) - 1
            plsc.store_scatter(cidx, [pos], flat, mask=inr)
            plsc.store_scatter(cvals, [pos], vals, mask=inr)
            cnt = cnt + jnp.sum(inr.astype(jnp.int32))
        return cnt

    cnt = lax.fori_loop(0, NBATCH, scan, jnp.int32(0))
    for k in range(8):
        posp = cnt + k * 16 + lane
        plsc.store_scatter(cidx, [posp], jnp.zeros((16,), jnp.int32))
        plsc.store_scatter(cvals, [posp], jnp.zeros((16,), jnp.float32))
    nb = (cnt + 127) // 128

    def fire(b, c):
        pltpu.async_copy(cvals.at[pl.ds(b * 128, 128)],
                         chunk.at[cidx.at[pl.ds(b * 128, 128)]], sem, add=True)
        return c

    lax.fori_loop(0, nb, fire, 0)

    def drain(b, c):
        pltpu.make_async_copy(cvals.at[pl.ds(b * 128, 128)],
                              chunk.at[cidx.at[pl.ds(b * 128, 128)]],
                              sem).wait()
        return c

    lax.fori_loop(0, nb, drain, 0)


def _sc_accumulate(node_hbm, edge_hbm, out_hbm, nbuf, ebuf, cidx, cvals,
                   zbuf, chunk, sem, value_fn):
    cid = lax.axis_index("c")
    sid = lax.axis_index("s")
    pltpu.sync_copy(node_hbm.at[pl.ds(sid * EPT, EPT)], nbuf.at[pl.ds(0, EPT)])
    pltpu.sync_copy(edge_hbm.at[pl.ds(sid * EPT, EPT)], ebuf.at[pl.ds(0, EPT)])
    for i in range((PADEPT - EPT) // 16):
        nbuf[pl.ds(EPT + i * 16, 16)] = jnp.zeros((16,), jnp.int32)
        ebuf[pl.ds(EPT + i * 16, 16)] = jnp.zeros((16,), jnp.int32)
    for rnd in range(NCHUNK // 2):
        chunk_id = rnd * 2 + cid
        node_lo = chunk_id * CROWS
        _sc_zero_stripe(zbuf, chunk, sid)
        plsc.subcore_barrier()
        _sc_scan_round(nbuf, ebuf, cidx, cvals, chunk, sem, node_lo, value_fn)
        plsc.subcore_barrier()

        @pl.when(sid < NOUT)
        def _():
            pltpu.sync_copy(
                chunk.at[pl.ds(sid * OSTRIPE, OSTRIPE)],
                out_hbm.at[pl.ds(chunk_id * CELEMS + sid * OSTRIPE, OSTRIPE)])

        plsc.subcore_barrier()


def _sc_mesh():
    return plsc.VectorSubcoreMesh(core_axis_name="c", subcore_axis_name="s")


def _sc_count_body(node_hbm, edge_hbm, out_hbm, nbuf, ebuf, cidx, cvals,
                   zbuf, chunk, sem):
    def ones(nv, ev):
        return jnp.ones((16,), jnp.float32)

    _sc_accumulate(node_hbm, edge_hbm, out_hbm, nbuf, ebuf, cidx, cvals,
                   zbuf, chunk, sem, ones)


def _sc_count(node, edge):
    k = pl.kernel(
        _sc_count_body,
        mesh=_sc_mesh(),
        out_type=jax.ShapeDtypeStruct((N * M,), jnp.float32),
        compiler_params=pltpu.CompilerParams(needs_layout_passes=False),
        scratch_types=[
            pltpu.VMEM((PADEPT,), jnp.int32),
            pltpu.VMEM((PADEPT,), jnp.int32),
            pltpu.VMEM((NBG * 128,), jnp.int32),
            pltpu.VMEM((NBG * 128,), jnp.float32),
            pltpu.VMEM((2000,), jnp.float32),
            pltpu.VMEM_SHARED((CELEMS,), jnp.float32),
            pltpu.SemaphoreType.DMA,
        ],
    )
    return k(node, edge)


def _sc_expw_body(node_hbm, edge_hbm, ns_hbm, c_hbm, es_hbm, out_hbm,
                  nbuf, ebuf, cidx, cvals, zbuf, chunk, ns_t, c_t, es_t, sem):
    pltpu.sync_copy(ns_hbm, ns_t)
    pltpu.sync_copy(c_hbm, c_t)
    pltpu.sync_copy(es_hbm, es_t)

    def expw(nv, ev):
        nsv = plsc.load_gather(ns_t, [nv])
        cv = plsc.load_gather(c_t, [nv])
        esv = plsc.load_gather(es_t, [ev])
        s = nsv + esv
        a = jnp.where(s > 0, s, NEG_SLOPE * s)
        return jnp.exp(a - cv)

    _sc_accumulate(node_hbm, edge_hbm, out_hbm, nbuf, ebuf, cidx, cvals,
                   zbuf, chunk, sem, expw)


def _sc_expw(node, edge, ns, c, es):
    k = pl.kernel(
        _sc_expw_body,
        mesh=_sc_mesh(),
        out_type=jax.ShapeDtypeStruct((N * M,), jnp.float32),
        compiler_params=pltpu.CompilerParams(needs_layout_passes=False),
        scratch_types=[
            pltpu.VMEM((PADEPT,), jnp.int32),
            pltpu.VMEM((PADEPT,), jnp.int32),
            pltpu.VMEM((NBG * 128,), jnp.int32),
            pltpu.VMEM((NBG * 128,), jnp.float32),
            pltpu.VMEM((2000,), jnp.float32),
            pltpu.VMEM_SHARED((CELEMS,), jnp.float32),
            pltpu.VMEM((N,), jnp.float32),
            pltpu.VMEM((N,), jnp.float32),
            pltpu.VMEM((M,), jnp.float32),
            pltpu.SemaphoreType.DMA,
        ],
    )
    return k(node, edge, ns, c, es)


def kernel(x, hyperedge_index, weight, att):
    node = hyperedge_index[0].astype(jnp.int32)
    edge = hyperedge_index[1].astype(jnp.int32)
    x2d = x[0]
    att1 = att[0, :, :D]
    att2 = att[0, :, D:]

    xw, ns3 = _tc_xw(x2d, weight, att1)
    ct = _sc_count(node, edge).reshape(N, M)
    esum, de8, dn3, s1 = _tc_esum(ct, xw)
    es2, c3, bn2, lh, s2 = _tc_small(ct, ns3, esum, de8, att2)
    et = _sc_expw(node, edge, ns3.reshape(N), c3.reshape(N),
                  es2.reshape(M)).reshape(N, M)
    oute = _tc_oute(et, xw, bn2.reshape(M, 1))
    outn = _tc_outn(et, dn3.reshape(N, 1), oute)

    constrain = jnp.abs((s1[0, 0] - s2[0, 0]) / float(E * D)) + lh[0, 0]
    return outn[None], constrain
